# baseline (speedup 1.0000x reference)
import jax
import jax.numpy as jnp
from jax.experimental import pallas as pl
from jax.experimental.pallas import tpu as pltpu

N = 10000
D_IN = 128
D_OUT = 128
BM = 200

_NT = (((1,), (1,)), ((), ()))


def _gcn_block(x_ref, adj_ref, w1_ref, b1_ref, w2_ref, b2_ref, o_ref):
    h = jnp.dot(adj_ref[...], x_ref[...], preferred_element_type=jnp.float32)
    h = (
        jax.lax.dot_general(
            h, w1_ref[...], _NT, preferred_element_type=jnp.float32
        )
        + b1_ref[...]
    )
    h = 0.5 * h * (1.0 + jax.lax.erf(h * 0.7071067811865476))
    o_ref[...] = (
        jax.lax.dot_general(
            h, w2_ref[...], _NT, preferred_element_type=jnp.float32
        )
        + b2_ref[...]
    )


def kernel(input, adj, W1, b1, W2, b2):
    b1r = b1.reshape(1, D_OUT)
    b2r = b2.reshape(1, D_OUT)
    grid = (N // BM,)
    return pl.pallas_call(
        _gcn_block,
        grid=grid,
        in_specs=[
            pl.BlockSpec((N, D_IN), lambda i: (0, 0)),
            pl.BlockSpec((BM, N), lambda i: (i, 0)),
            pl.BlockSpec((D_OUT, D_IN), lambda i: (0, 0)),
            pl.BlockSpec((1, D_OUT), lambda i: (0, 0)),
            pl.BlockSpec((D_OUT, D_OUT), lambda i: (0, 0)),
            pl.BlockSpec((1, D_OUT), lambda i: (0, 0)),
        ],
        out_specs=pl.BlockSpec((BM, D_OUT), lambda i: (i, 0)),
        out_shape=jax.ShapeDtypeStruct((N, D_OUT), jnp.float32),
        compiler_params=pltpu.CompilerParams(
            dimension_semantics=("arbitrary",),
            vmem_limit_bytes=128 * 1024 * 1024,
        ),
    )(input, adj, W1, b1r, W2, b2r)


# speedup vs baseline: 1.0503x; 1.0503x over previous
import jax
import jax.numpy as jnp
from jax.experimental import pallas as pl
from jax.experimental.pallas import tpu as pltpu

N = 10000
D_IN = 128
D_OUT = 128
BM = 400

_NT = (((1,), (1,)), ((), ()))


def _gcn_block(x_ref, adj_ref, w1_ref, b1_ref, w2_ref, b2_ref, o_ref):
    h = jnp.dot(adj_ref[...], x_ref[...], preferred_element_type=jnp.float32)
    h = (
        jax.lax.dot_general(
            h, w1_ref[...], _NT, preferred_element_type=jnp.float32
        )
        + b1_ref[...]
    )
    h = 0.5 * h * (1.0 + jax.lax.erf(h * 0.7071067811865476))
    o_ref[...] = (
        jax.lax.dot_general(
            h, w2_ref[...], _NT, preferred_element_type=jnp.float32
        )
        + b2_ref[...]
    )


def kernel(input, adj, W1, b1, W2, b2):
    b1r = b1.reshape(1, D_OUT)
    b2r = b2.reshape(1, D_OUT)
    grid = (N // BM,)
    return pl.pallas_call(
        _gcn_block,
        grid=grid,
        in_specs=[
            pl.BlockSpec((N, D_IN), lambda i: (0, 0)),
            pl.BlockSpec((BM, N), lambda i: (i, 0)),
            pl.BlockSpec((D_OUT, D_IN), lambda i: (0, 0)),
            pl.BlockSpec((1, D_OUT), lambda i: (0, 0)),
            pl.BlockSpec((D_OUT, D_OUT), lambda i: (0, 0)),
            pl.BlockSpec((1, D_OUT), lambda i: (0, 0)),
        ],
        out_specs=pl.BlockSpec((BM, D_OUT), lambda i: (i, 0)),
        out_shape=jax.ShapeDtypeStruct((N, D_OUT), jnp.float32),
        compiler_params=pltpu.CompilerParams(
            dimension_semantics=("parallel",),
            vmem_limit_bytes=128 * 1024 * 1024,
        ),
    )(input, adj, W1, b1r, W2, b2r)
